# baseline (speedup 1.0000x reference)
import jax
import jax.numpy as jnp
from jax import lax
from jax.experimental import pallas as pl
from jax.experimental.pallas import tpu as pltpu
from jax.experimental.pallas import tpu_sc as plsc

_NC = 2
_NS = 16
_NW = _NC * _NS
_LANES = 16
_CHUNK = 8
_NSLOT = 4
_GRP = 24


def kernel(token, token_table, pos_table):
    b, s = token.shape
    v, d = token_table.shape
    n = b * s
    pos_span = s // _NW
    n_blocks = pos_span // _CHUNK
    n_groups = n_blocks // _NSLOT
    n_vec = d // _LANES

    def body(tok_hbm, table_hbm, pos_hbm, out_hbm, idx_all, *rest):
        bufs = [rest[p * (b + 1): p * (b + 1) + b] for p in range(_NSLOT)]
        posb = [rest[p * (b + 1) + b] for p in range(_NSLOT)]
        sems = rest[_NSLOT * (b + 1):-1]
        gsem = sems[0::3]
        psem = sems[1::3]
        ssem = sems[2::3]
        isem = rest[-1]

        wid = lax.axis_index("s") * _NC + lax.axis_index("c")
        pbase = wid * pos_span

        preloads = [
            pltpu.async_copy(tok_hbm.at[bi, pl.ds(pbase, pos_span)],
                             idx_all.at[pl.ds(bi * pos_span, pos_span)],
                             isem)
            for bi in range(b)
        ]
        for c in preloads:
            c.wait()

        def issue_block(h, p):
            for bi in range(b):
                idx = idx_all.at[pl.ds(bi * pos_span + h * _CHUNK, _CHUNK)]
                pltpu.async_copy(table_hbm.at[idx], bufs[p][bi], gsem[p])
            pltpu.async_copy(pos_hbm.at[pl.ds(pbase + h * _CHUNK, _CHUNK)],
                             posb[p], psem[p])

        def issue_stores(h, p):
            for bi in range(b):
                off = bi * s + pbase + h * _CHUNK
                pltpu.async_copy(bufs[p][bi], out_hbm.at[pl.ds(off, _CHUNK)],
                                 ssem[p])

        def wait_gathers(p):
            idx0 = idx_all.at[pl.ds(0, _CHUNK)]
            for bi in range(b):
                pltpu.make_async_copy(table_hbm.at[idx0], bufs[p][bi],
                                      gsem[p]).wait()
            pltpu.make_async_copy(pos_hbm.at[pl.ds(0, _CHUNK)], posb[p],
                                  psem[p]).wait()

        def wait_stores(p):
            for bi in range(b):
                pltpu.make_async_copy(bufs[p][bi],
                                      out_hbm.at[pl.ds(0, _CHUNK)],
                                      ssem[p]).wait()

        def fuse(p):
            @plsc.parallel_loop(0, _CHUNK)
            def row_body(r, _p=p):
                for j0 in range(0, n_vec, _GRP):
                    vals = [posb[_p][r, pl.ds((j0 + j) * _LANES, _LANES)]
                            for j in range(_GRP)]
                    for bi in range(b):
                        for j in range(_GRP):
                            sl = pl.ds((j0 + j) * _LANES, _LANES)
                            plsc.addupdate(bufs[_p][bi].at[r, sl], vals[j])

        for h in range(_NSLOT - 1):
            issue_block(h, h)

        def group(g, carry):
            for p in range(_NSLOT):
                h = g * _NSLOT + p
                q = (p + _NSLOT - 1) % _NSLOT
                if p == 0:
                    @pl.when(g > 0)
                    def _():
                        wait_stores(q)
                    issue_block(h + _NSLOT - 1, q)
                else:
                    @pl.when(g < n_groups - 1)
                    def _():
                        wait_stores(q)
                        issue_block(h + _NSLOT - 1, q)
                wait_gathers(p)
                fuse(p)
                issue_stores(h, p)
            return carry

        lax.fori_loop(0, n_groups, group, 0)

        for p in range(_NSLOT):
            wait_stores(p)

    mesh = plsc.VectorSubcoreMesh(core_axis_name="c", subcore_axis_name="s")
    scratch = [pltpu.VMEM((b * pos_span,), jnp.int32)]
    for _ in range(_NSLOT):
        scratch += [pltpu.VMEM((_CHUNK, d), jnp.float32) for _ in range(b)]
        scratch += [pltpu.VMEM((_CHUNK, d), jnp.float32)]
    scratch += [pltpu.SemaphoreType.DMA for _ in range(3 * _NSLOT + 1)]
    run = pl.kernel(
        body,
        mesh=mesh,
        out_type=jax.ShapeDtypeStruct((n, d), jnp.float32),
        scratch_types=scratch,
    )
    out = run(token, token_table, pos_table)
    return out.reshape(b, s, d)
